# baseline (speedup 1.0000x reference)
import jax
import jax.numpy as jnp
from jax.experimental import pallas as pl

K_EIG, K_HOPS, ALPHA, LAP_EPS = 16, 2, 0.05, 1e-4


def _layer_norm(x, g, b):
    mu = jnp.mean(x, axis=-1, keepdims=True)
    var = jnp.mean((x - mu) ** 2, axis=-1, keepdims=True)
    return (x - mu) / jnp.sqrt(var + 1e-5) * g + b


def _gcn_norm(A):
    deg = jnp.sum(A, axis=1)
    dis = jnp.where(deg > 0, 1.0 / jnp.sqrt(jnp.maximum(deg, 1e-12)), 0.0)
    return dis[:, :, None] * A * dis[:, None, :]


def _ssg_conv(x, An, w, b):
    h = ALPHA * x
    cur = x
    for _ in range(K_HOPS):
        cur = jnp.einsum('grc,grd->gcd', An, cur)
        h = h + (1.0 - ALPHA) / K_HOPS * cur
    return h @ w + b


def _stable_laplacian(A):
    deg = jnp.sum(A, axis=-1)
    dis = 1.0 / jnp.sqrt(jnp.maximum(deg, LAP_EPS))
    I = jnp.eye(A.shape[-1], dtype=A.dtype)
    L = I - dis[..., :, None] * A * dis[..., None, :]
    L = 0.5 * (L + jnp.swapaxes(L, -1, -2)) + LAP_EPS * I
    return L


def _identity_kernel(x_ref, o_ref):
    o_ref[...] = x_ref[...]


def kernel(features, adjacency, et_w, et_b, lin0_w, lin0_b, lin1_w, lin1_b, ln0_g, ln0_b, ln1_g, ln1_b):
    Bz, Nn, Tt, Dd = features.shape
    x = jnp.transpose(features, (0, 2, 1, 3)).reshape(-1, Nn, Dd)
    A = adjacency.reshape(-1, Nn, Nn)
    mask = A > 0
    Aw = jnp.where(mask, jax.nn.softplus(A * et_w[0, 0] + et_b[0]), 0.0)
    An = _gcn_norm(Aw)
    L = _stable_laplacian(A)
    _, eigvecs = jnp.linalg.eigh(L)
    pe = eigvecs[..., :K_EIG]
    signs = jnp.sign(jnp.sum(pe, axis=-2, keepdims=True))
    signs = jnp.where(signs == 0, jnp.ones_like(signs), signs)
    pe = pe * signs
    pe = jnp.nan_to_num(pe, nan=0.0, posinf=1.0, neginf=-1.0)
    h = jnp.concatenate([x, pe], axis=-1)
    h = _layer_norm(_ssg_conv(h, An, lin0_w, lin0_b), ln0_g, ln0_b)
    out = _layer_norm(_ssg_conv(h, An, lin1_w, lin1_b), ln1_g, ln1_b)
    h = out + h
    h = jnp.transpose(h.reshape(Bz, Tt, Nn, Dd), (0, 2, 1, 3))
    return pl.pallas_call(
        _identity_kernel,
        grid=(Bz,),
        in_specs=[pl.BlockSpec((1, Nn, Tt, Dd), lambda i: (i, 0, 0, 0))],
        out_specs=pl.BlockSpec((1, Nn, Tt, Dd), lambda i: (i, 0, 0, 0)),
        out_shape=jax.ShapeDtypeStruct(h.shape, h.dtype),
    )(h)


# speedup vs baseline: 95.7281x; 95.7281x over previous
import jax
import jax.numpy as jnp
from jax.experimental import pallas as pl
from jax.experimental.pallas import tpu as pltpu

_N = 19
_K_EIG = 16
_ALPHA = 0.05
_BETA = (1.0 - _ALPHA) / 2.0
_LAP_EPS = 1e-4
_NSWEEPS = 6
_PAIRS = tuple((p, q) for p in range(_N - 1) for q in range(p + 1, _N))


def _softplus(z):
    return jnp.maximum(z, 0.0) + jnp.log1p(jnp.exp(-jnp.abs(z)))


def _mixer_body(et_ref, A_ref, X_ref, w0_ref, w1_ref, b0_ref, b1_ref,
                g0_ref, be0_ref, g1_ref, be1_ref, out_ref, L_ref, V_ref):
    N = _N
    tile = A_ref.shape[2:]

    A = A_ref[...]
    etw = et_ref[0]
    etb = et_ref[1]

    Aw = jnp.where(A > 0.0, _softplus(A * etw + etb), 0.0)
    degc = jnp.sum(Aw, axis=0)
    disc = jnp.where(degc > 0.0, jax.lax.rsqrt(jnp.maximum(degc, 1e-12)), 0.0)
    An = disc[:, None] * Aw * disc[None, :]

    degr = jnp.sum(A, axis=1)
    disl = jax.lax.rsqrt(jnp.maximum(degr, _LAP_EPS))
    S = disl[:, None] * A * disl[None, :]
    L_ref[...] = -0.5 * (S + jnp.swapaxes(S, 0, 1))
    V_ref[...] = jnp.zeros_like(S)
    for i in range(N):
        L_ref[i, i] = (1.0 + _LAP_EPS) - S[i, i]
        V_ref[i, i] = jnp.ones(tile, jnp.float32)

    def sweep(_, carry):
        for (p, q) in _PAIRS:
            app = L_ref[p, p]
            aqq = L_ref[q, q]
            apq = L_ref[p, q]
            denom = 2.0 * apq
            safe = jnp.abs(denom) > 1e-37
            tau = (aqq - app) / jnp.where(safe, denom, 1.0)
            tau = jnp.clip(tau, -1e18, 1e18)
            tnum = jnp.where(tau >= 0.0, 1.0, -1.0)
            t = tnum / (jnp.abs(tau) + jnp.sqrt(1.0 + tau * tau))
            t = jnp.where(safe, t, 0.0)
            c = jax.lax.rsqrt(1.0 + t * t)
            s = t * c
            rp = L_ref[p, :]
            rq = L_ref[q, :]
            L_ref[p, :] = c * rp - s * rq
            L_ref[q, :] = s * rp + c * rq
            cp = L_ref[:, p]
            cq = L_ref[:, q]
            L_ref[:, p] = c * cp - s * cq
            L_ref[:, q] = s * cp + c * cq
            vp = V_ref[:, p]
            vq = V_ref[:, q]
            V_ref[:, p] = c * vp - s * vq
            V_ref[:, q] = s * vp + c * vq
        return carry

    jax.lax.fori_loop(0, _NSWEEPS, sweep, 0)

    lam = [L_ref[i, i] for i in range(N)]
    ranks = []
    for i in range(N):
        r = jnp.zeros(tile, jnp.float32)
        for j in range(N):
            if j == i:
                continue
            if j < i:
                r += jnp.where((lam[j] < lam[i]) | (lam[j] == lam[i]), 1.0, 0.0)
            else:
                r += jnp.where(lam[j] < lam[i], 1.0, 0.0)
        ranks.append(r)

    vcols = [V_ref[:, i] for i in range(N)]
    pes = []
    for k in range(_K_EIG):
        acc = jnp.zeros((N,) + tile, jnp.float32)
        for i in range(N):
            acc += jnp.where(ranks[i] == float(k), 1.0, 0.0) * vcols[i]
        ssum = jnp.sum(acc, axis=0)
        sgn = jnp.where(ssum < 0.0, -1.0, 1.0)
        acc = acc * sgn
        acc = jnp.where(jnp.isnan(acc), 0.0, acc)
        pes.append(acc)

    def prop(Z):
        acc = An[0][:, None] * Z[0][None, :]
        for r in range(1, N):
            acc += An[r][:, None] * Z[r][None, :]
        return acc

    def ssg(Z, bref):
        p1 = prop(Z)
        p2 = prop(p1)
        return _ALPHA * Z + _BETA * p1 + _BETA * p2 + bref[...][None]

    def ln(u, gref, bref):
        mu = jnp.mean(u, axis=1, keepdims=True)
        d = u - mu
        var = jnp.mean(d * d, axis=1, keepdims=True)
        return d / jnp.sqrt(var + 1e-5) * gref[...][None] + bref[...][None]

    X = X_ref[...]
    w0 = w0_ref[...]
    D = X.shape[1]
    z0 = X[:, 0][:, None] * w0[0][None]
    for i in range(1, D):
        z0 += X[:, i][:, None] * w0[i][None]
    for i in range(_K_EIG):
        z0 += pes[i][:, None] * w0[D + i][None]
    h0 = ln(ssg(z0, b0_ref), g0_ref, be0_ref)

    w1 = w1_ref[...]
    z1 = h0[:, 0][:, None] * w1[0][None]
    for i in range(1, D):
        z1 += h0[:, i][:, None] * w1[i][None]
    out_ref[...] = ln(ssg(z1, b1_ref), g1_ref, be1_ref) + h0


def _forward(features, adjacency, et_w, et_b, lin0_w, lin0_b, lin1_w, lin1_b,
             ln0_g, ln0_b, ln1_g, ln1_b, interpret=False):
    Bz, Nn, Tt, Dd = features.shape
    G = Bz * Tt
    x = jnp.transpose(features, (0, 2, 1, 3)).reshape(G, Nn, Dd)
    A = adjacency.reshape(G, Nn, Nn)
    chunks = G // 128
    sub = 8
    while chunks % sub:
        sub //= 2
    grid = chunks // sub
    At = jnp.transpose(A, (1, 2, 0)).reshape(Nn, Nn, chunks, 128)
    Xt = jnp.transpose(x, (1, 2, 0)).reshape(Nn, Dd, chunks, 128)
    in0 = Dd + _K_EIG
    w0b = jnp.broadcast_to(lin0_w.reshape(in0, Dd, 1, 1), (in0, Dd, 1, 128))
    w1b = jnp.broadcast_to(lin1_w.reshape(Dd, Dd, 1, 1), (Dd, Dd, 1, 128))

    def vecb(v):
        return jnp.broadcast_to(v.reshape(Dd, 1, 1), (Dd, 1, 128))

    etv = jnp.concatenate([et_w.reshape(-1), et_b.reshape(-1)]).astype(jnp.float32)

    def cspec(shp):
        nd = len(shp)
        return pl.BlockSpec(shp, lambda i, _n=nd: (0,) * _n)

    out = pl.pallas_call(
        _mixer_body,
        grid=(grid,),
        in_specs=[
            pl.BlockSpec(memory_space=pltpu.SMEM),
            pl.BlockSpec((Nn, Nn, sub, 128), lambda i: (0, 0, i, 0)),
            pl.BlockSpec((Nn, Dd, sub, 128), lambda i: (0, 0, i, 0)),
            cspec((in0, Dd, 1, 128)),
            cspec((Dd, Dd, 1, 128)),
            cspec((Dd, 1, 128)),
            cspec((Dd, 1, 128)),
            cspec((Dd, 1, 128)),
            cspec((Dd, 1, 128)),
            cspec((Dd, 1, 128)),
            cspec((Dd, 1, 128)),
        ],
        out_specs=pl.BlockSpec((Nn, Dd, sub, 128), lambda i: (0, 0, i, 0)),
        out_shape=jax.ShapeDtypeStruct((Nn, Dd, chunks, 128), jnp.float32),
        scratch_shapes=[
            pltpu.VMEM((Nn, Nn, sub, 128), jnp.float32),
            pltpu.VMEM((Nn, Nn, sub, 128), jnp.float32),
        ],
        interpret=interpret,
    )(etv, At, Xt, w0b, w1b, vecb(lin0_b), vecb(lin1_b),
      vecb(ln0_g), vecb(ln0_b), vecb(ln1_g), vecb(ln1_b))

    h = jnp.transpose(out.reshape(Nn, Dd, G), (2, 0, 1))
    return jnp.transpose(h.reshape(Bz, Tt, Nn, Dd), (0, 2, 1, 3))


def kernel(features, adjacency, et_w, et_b, lin0_w, lin0_b, lin1_w, lin1_b,
           ln0_g, ln0_b, ln1_g, ln1_b):
    return _forward(features, adjacency, et_w, et_b, lin0_w, lin0_b,
                    lin1_w, lin1_b, ln0_g, ln0_b, ln1_g, ln1_b)


# speedup vs baseline: 110.0727x; 1.1498x over previous
import jax
import jax.numpy as jnp
from jax.experimental import pallas as pl
from jax.experimental.pallas import tpu as pltpu

_N = 19
_K_EIG = 16
_ALPHA = 0.05
_BETA = (1.0 - _ALPHA) / 2.0
_LAP_EPS = 1e-4
_NSWEEPS = 6
_PAIRS = tuple((p, q) for p in range(_N - 1) for q in range(p + 1, _N))


def _softplus(z):
    return jnp.maximum(z, 0.0) + jnp.log1p(jnp.exp(-jnp.abs(z)))


def _mixer_body(et_ref, A_ref, X_ref, w0_ref, w1_ref, b0_ref, b1_ref,
                g0_ref, be0_ref, g1_ref, be1_ref, out_ref, L_ref, V_ref):
    N = _N
    tile = A_ref.shape[2:]

    A = A_ref[...]
    etw = et_ref[0]
    etb = et_ref[1]

    Aw = jnp.where(A > 0.0, _softplus(A * etw + etb), 0.0)
    degc = jnp.sum(Aw, axis=0)
    disc = jnp.where(degc > 0.0, jax.lax.rsqrt(jnp.maximum(degc, 1e-12)), 0.0)
    An = disc[:, None] * Aw * disc[None, :]

    degr = jnp.sum(A, axis=1)
    disl = jax.lax.rsqrt(jnp.maximum(degr, _LAP_EPS))
    S = disl[:, None] * A * disl[None, :]
    L_ref[...] = -0.5 * (S + jnp.swapaxes(S, 0, 1))
    V_ref[...] = jnp.zeros_like(S)
    for i in range(N):
        L_ref[i, i] = (1.0 + _LAP_EPS) - S[i, i]
        V_ref[i, i] = jnp.ones(tile, jnp.float32)

    def sweep(_, carry):
        for (p, q) in _PAIRS:
            app = L_ref[p, p]
            aqq = L_ref[q, q]
            apq = L_ref[p, q]
            denom = 2.0 * apq
            safe = jnp.abs(denom) > 1e-37
            tau = (aqq - app) / jnp.where(safe, denom, 1.0)
            tau = jnp.clip(tau, -1e18, 1e18)
            tnum = jnp.where(tau >= 0.0, 1.0, -1.0)
            t = tnum / (jnp.abs(tau) + jnp.sqrt(1.0 + tau * tau))
            t = jnp.where(safe, t, 0.0)
            c = jax.lax.rsqrt(1.0 + t * t)
            s = t * c
            rp = L_ref[p, :]
            rq = L_ref[q, :]
            L_ref[p, :] = c * rp - s * rq
            L_ref[q, :] = s * rp + c * rq
            cp = L_ref[:, p]
            cq = L_ref[:, q]
            L_ref[:, p] = c * cp - s * cq
            L_ref[:, q] = s * cp + c * cq
            vp = V_ref[:, p]
            vq = V_ref[:, q]
            V_ref[:, p] = c * vp - s * vq
            V_ref[:, q] = s * vp + c * vq
        return carry

    jax.lax.fori_loop(0, _NSWEEPS, sweep, 0)

    lam = [L_ref[i, i] for i in range(N)]
    ranks = []
    for i in range(N):
        r = jnp.zeros(tile, jnp.float32)
        for j in range(N):
            if j == i:
                continue
            if j < i:
                r += jnp.where((lam[j] < lam[i]) | (lam[j] == lam[i]), 1.0, 0.0)
            else:
                r += jnp.where(lam[j] < lam[i], 1.0, 0.0)
        ranks.append(r)

    vcols = [V_ref[:, i] for i in range(N)]
    pes = []
    for k in range(_K_EIG):
        acc = jnp.zeros((N,) + tile, jnp.float32)
        for i in range(N):
            acc += jnp.where(ranks[i] == float(k), 1.0, 0.0) * vcols[i]
        ssum = jnp.sum(acc, axis=0)
        sgn = jnp.where(ssum < 0.0, -1.0, 1.0)
        acc = acc * sgn
        acc = jnp.where(jnp.isnan(acc), 0.0, acc)
        pes.append(acc)

    mcols = []
    for r in range(N):
        acc = An[0] * An[r, 0]
        for m in range(1, N):
            acc += An[m] * An[r, m]
        mcols.append(_BETA * (An[r] + acc))

    def ssg(Z, bref):
        mv = mcols[0][:, None] * Z[0][None, :]
        for r in range(1, N):
            mv += mcols[r][:, None] * Z[r][None, :]
        return _ALPHA * Z + mv + bref[...][None]

    def ln(u, gref, bref):
        mu = jnp.mean(u, axis=1, keepdims=True)
        d = u - mu
        var = jnp.mean(d * d, axis=1, keepdims=True)
        return d / jnp.sqrt(var + 1e-5) * gref[...][None] + bref[...][None]

    X = X_ref[...]
    w0 = w0_ref[...]
    D = X.shape[1]
    z0 = X
    for i in range(_K_EIG):
        z0 += pes[i][:, None] * w0[i][None]
    h0 = ln(ssg(z0, b0_ref), g0_ref, be0_ref)

    w1 = w1_ref[...]
    z1 = h0[:, 0][:, None] * w1[0][None]
    for i in range(1, D):
        z1 += h0[:, i][:, None] * w1[i][None]
    out_ref[...] = ln(ssg(z1, b1_ref), g1_ref, be1_ref) + h0


def _xform_body(x_ref, w_ref, o_ref):
    o_ref[...] = jnp.dot(x_ref[...], w_ref[...],
                         preferred_element_type=jnp.float32)


def _forward(features, adjacency, et_w, et_b, lin0_w, lin0_b, lin1_w, lin1_b,
             ln0_g, ln0_b, ln1_g, ln1_b, interpret=False):
    Bz, Nn, Tt, Dd = features.shape
    G = Bz * Tt
    x = jnp.transpose(features, (0, 2, 1, 3)).reshape(G, Nn, Dd)
    A = adjacency.reshape(G, Nn, Nn)
    chunks = G // 128
    sub = 8
    while chunks % sub:
        sub //= 2
    grid = chunks // sub
    At = jnp.transpose(A, (1, 2, 0)).reshape(Nn, Nn, chunks, 128)

    rows = G * Nn
    rb = 2048
    while rows % rb:
        rb //= 2
    xf = x.reshape(rows, Dd)
    z0x = pl.pallas_call(
        _xform_body,
        grid=(rows // rb,),
        in_specs=[pl.BlockSpec((rb, Dd), lambda i: (i, 0)),
                  pl.BlockSpec((Dd, Dd), lambda i: (0, 0))],
        out_specs=pl.BlockSpec((rb, Dd), lambda i: (i, 0)),
        out_shape=jax.ShapeDtypeStruct((rows, Dd), jnp.float32),
        interpret=interpret,
    )(xf, lin0_w[:Dd])

    Xt = jnp.transpose(z0x.reshape(G, Nn, Dd), (1, 2, 0)).reshape(
        Nn, Dd, chunks, 128)
    w0b = jnp.broadcast_to(lin0_w[Dd:].reshape(_K_EIG, Dd, 1, 1),
                           (_K_EIG, Dd, 1, 128))
    w1b = jnp.broadcast_to(lin1_w.reshape(Dd, Dd, 1, 1), (Dd, Dd, 1, 128))

    def vecb(v):
        return jnp.broadcast_to(v.reshape(Dd, 1, 1), (Dd, 1, 128))

    etv = jnp.concatenate([et_w.reshape(-1), et_b.reshape(-1)]).astype(jnp.float32)

    def cspec(shp):
        nd = len(shp)
        return pl.BlockSpec(shp, lambda i, _n=nd: (0,) * _n)

    out = pl.pallas_call(
        _mixer_body,
        grid=(grid,),
        in_specs=[
            pl.BlockSpec(memory_space=pltpu.SMEM),
            pl.BlockSpec((Nn, Nn, sub, 128), lambda i: (0, 0, i, 0)),
            pl.BlockSpec((Nn, Dd, sub, 128), lambda i: (0, 0, i, 0)),
            cspec((_K_EIG, Dd, 1, 128)),
            cspec((Dd, Dd, 1, 128)),
            cspec((Dd, 1, 128)),
            cspec((Dd, 1, 128)),
            cspec((Dd, 1, 128)),
            cspec((Dd, 1, 128)),
            cspec((Dd, 1, 128)),
            cspec((Dd, 1, 128)),
        ],
        out_specs=pl.BlockSpec((Nn, Dd, sub, 128), lambda i: (0, 0, i, 0)),
        out_shape=jax.ShapeDtypeStruct((Nn, Dd, chunks, 128), jnp.float32),
        scratch_shapes=[
            pltpu.VMEM((Nn, Nn, sub, 128), jnp.float32),
            pltpu.VMEM((Nn, Nn, sub, 128), jnp.float32),
        ],
        interpret=interpret,
    )(etv, At, Xt, w0b, w1b, vecb(lin0_b), vecb(lin1_b),
      vecb(ln0_g), vecb(ln0_b), vecb(ln1_g), vecb(ln1_b))

    h = jnp.transpose(out.reshape(Nn, Dd, G), (2, 0, 1))
    return jnp.transpose(h.reshape(Bz, Tt, Nn, Dd), (0, 2, 1, 3))


def kernel(features, adjacency, et_w, et_b, lin0_w, lin0_b, lin1_w, lin1_b,
           ln0_g, ln0_b, ln1_g, ln1_b):
    return _forward(features, adjacency, et_w, et_b, lin0_w, lin0_b,
                    lin1_w, lin1_b, ln0_g, ln0_b, ln1_g, ln1_b)


# speedup vs baseline: 120.5348x; 1.0950x over previous
import jax
import jax.numpy as jnp
from jax.experimental import pallas as pl
from jax.experimental.pallas import tpu as pltpu

_N = 19
_K_EIG = 16
_ALPHA = 0.05
_BETA = (1.0 - _ALPHA) / 2.0
_LAP_EPS = 1e-4
_NSWEEPS = 6
_PAIRS = tuple((p, q) for p in range(_N - 1) for q in range(p + 1, _N))


def _softplus(z):
    return jnp.maximum(z, 0.0) + jnp.log1p(jnp.exp(-jnp.abs(z)))


def _mixer_body(et_ref, A_ref, X_ref, w0_ref, w1_ref, b0_ref, b1_ref,
                g0_ref, be0_ref, g1_ref, be1_ref, out_ref, L_ref, V_ref):
    N = _N
    tile = A_ref.shape[2:]

    A = A_ref[...]
    etw = et_ref[0]
    etb = et_ref[1]

    Aw = jnp.where(A > 0.0, _softplus(A * etw + etb), 0.0)
    degc = jnp.sum(Aw, axis=0)
    disc = jnp.where(degc > 0.0, jax.lax.rsqrt(jnp.maximum(degc, 1e-12)), 0.0)
    An = disc[:, None] * Aw * disc[None, :]

    degr = jnp.sum(A, axis=1)
    disl = jax.lax.rsqrt(jnp.maximum(degr, _LAP_EPS))
    S = disl[:, None] * A * disl[None, :]
    L_ref[...] = -0.5 * (S + jnp.swapaxes(S, 0, 1))
    V_ref[...] = jnp.zeros_like(S)
    for i in range(N):
        L_ref[i, i] = (1.0 + _LAP_EPS) - S[i, i]
        V_ref[i, i] = jnp.ones(tile, jnp.float32)

    def sweep(_, carry):
        for (p, q) in _PAIRS:
            cp = L_ref[:, p]
            cq = L_ref[:, q]
            app = cp[p]
            aqq = cq[q]
            apq = cp[q]
            denom = 2.0 * apq
            safe = jnp.abs(denom) > 1e-37
            tau = (aqq - app) / jnp.where(safe, denom, 1.0)
            tau = jnp.clip(tau, -1e18, 1e18)
            tnum = jnp.where(tau >= 0.0, 1.0, -1.0)
            t = tnum / (jnp.abs(tau) + jnp.sqrt(1.0 + tau * tau))
            t = jnp.where(safe, t, 0.0)
            c = jax.lax.rsqrt(1.0 + t * t)
            s = t * c
            ncp = c * cp - s * cq
            ncq = s * cp + c * cq
            L_ref[:, p] = ncp
            L_ref[:, q] = ncq
            L_ref[p, :] = ncp
            L_ref[q, :] = ncq
            zero = jnp.zeros_like(app)
            L_ref[p, p] = app - t * apq
            L_ref[q, q] = aqq + t * apq
            L_ref[p, q] = zero
            L_ref[q, p] = zero
            vp = V_ref[:, p]
            vq = V_ref[:, q]
            V_ref[:, p] = c * vp - s * vq
            V_ref[:, q] = s * vp + c * vq
        return carry

    jax.lax.fori_loop(0, _NSWEEPS, sweep, 0)

    lam = [L_ref[i, i] for i in range(N)]
    ranks = []
    for i in range(N):
        r = jnp.zeros(tile, jnp.float32)
        for j in range(N):
            if j == i:
                continue
            if j < i:
                r += jnp.where((lam[j] < lam[i]) | (lam[j] == lam[i]), 1.0, 0.0)
            else:
                r += jnp.where(lam[j] < lam[i], 1.0, 0.0)
        ranks.append(r)

    vcols = [V_ref[:, i] for i in range(N)]
    pes = []
    for k in range(_K_EIG):
        acc = jnp.zeros((N,) + tile, jnp.float32)
        for i in range(N):
            acc += jnp.where(ranks[i] == float(k), 1.0, 0.0) * vcols[i]
        ssum = jnp.sum(acc, axis=0)
        sgn = jnp.where(ssum < 0.0, -1.0, 1.0)
        acc = acc * sgn
        acc = jnp.where(jnp.isnan(acc), 0.0, acc)
        pes.append(acc)

    mcols = []
    for r in range(N):
        acc = An[0] * An[r, 0]
        for m in range(1, N):
            acc += An[m] * An[r, m]
        mcols.append(_BETA * (An[r] + acc))

    def ssg(Z, bref):
        mv = mcols[0][:, None] * Z[0][None, :]
        for r in range(1, N):
            mv += mcols[r][:, None] * Z[r][None, :]
        return _ALPHA * Z + mv + bref[...][None]

    def ln(u, gref, bref):
        mu = jnp.mean(u, axis=1, keepdims=True)
        d = u - mu
        var = jnp.mean(d * d, axis=1, keepdims=True)
        return d / jnp.sqrt(var + 1e-5) * gref[...][None] + bref[...][None]

    X = X_ref[...]
    w0 = w0_ref[...]
    D = X.shape[1]
    z0 = X
    for i in range(_K_EIG):
        z0 += pes[i][:, None] * w0[i][None]
    h0 = ln(ssg(z0, b0_ref), g0_ref, be0_ref)

    w1 = w1_ref[...]
    z1 = h0[:, 0][:, None] * w1[0][None]
    for i in range(1, D):
        z1 += h0[:, i][:, None] * w1[i][None]
    out_ref[...] = ln(ssg(z1, b1_ref), g1_ref, be1_ref) + h0


def _xform_body(x_ref, w_ref, o_ref):
    o_ref[...] = jnp.dot(x_ref[...], w_ref[...],
                         preferred_element_type=jnp.float32)


def _forward(features, adjacency, et_w, et_b, lin0_w, lin0_b, lin1_w, lin1_b,
             ln0_g, ln0_b, ln1_g, ln1_b, interpret=False):
    Bz, Nn, Tt, Dd = features.shape
    G = Bz * Tt
    x = jnp.transpose(features, (0, 2, 1, 3)).reshape(G, Nn, Dd)
    A = adjacency.reshape(G, Nn, Nn)
    chunks = G // 128
    sub = 8
    while chunks % sub:
        sub //= 2
    grid = chunks // sub
    At = jnp.transpose(A, (1, 2, 0)).reshape(Nn, Nn, chunks, 128)

    rows = G * Nn
    rb = 2048
    while rows % rb:
        rb //= 2
    xf = x.reshape(rows, Dd)
    z0x = pl.pallas_call(
        _xform_body,
        grid=(rows // rb,),
        in_specs=[pl.BlockSpec((rb, Dd), lambda i: (i, 0)),
                  pl.BlockSpec((Dd, Dd), lambda i: (0, 0))],
        out_specs=pl.BlockSpec((rb, Dd), lambda i: (i, 0)),
        out_shape=jax.ShapeDtypeStruct((rows, Dd), jnp.float32),
        interpret=interpret,
    )(xf, lin0_w[:Dd])

    Xt = jnp.transpose(z0x.reshape(G, Nn, Dd), (1, 2, 0)).reshape(
        Nn, Dd, chunks, 128)
    w0b = jnp.broadcast_to(lin0_w[Dd:].reshape(_K_EIG, Dd, 1, 1),
                           (_K_EIG, Dd, 1, 128))
    w1b = jnp.broadcast_to(lin1_w.reshape(Dd, Dd, 1, 1), (Dd, Dd, 1, 128))

    def vecb(v):
        return jnp.broadcast_to(v.reshape(Dd, 1, 1), (Dd, 1, 128))

    etv = jnp.concatenate([et_w.reshape(-1), et_b.reshape(-1)]).astype(jnp.float32)

    def cspec(shp):
        nd = len(shp)
        return pl.BlockSpec(shp, lambda i, _n=nd: (0,) * _n)

    out = pl.pallas_call(
        _mixer_body,
        grid=(grid,),
        in_specs=[
            pl.BlockSpec(memory_space=pltpu.SMEM),
            pl.BlockSpec((Nn, Nn, sub, 128), lambda i: (0, 0, i, 0)),
            pl.BlockSpec((Nn, Dd, sub, 128), lambda i: (0, 0, i, 0)),
            cspec((_K_EIG, Dd, 1, 128)),
            cspec((Dd, Dd, 1, 128)),
            cspec((Dd, 1, 128)),
            cspec((Dd, 1, 128)),
            cspec((Dd, 1, 128)),
            cspec((Dd, 1, 128)),
            cspec((Dd, 1, 128)),
            cspec((Dd, 1, 128)),
        ],
        out_specs=pl.BlockSpec((Nn, Dd, sub, 128), lambda i: (0, 0, i, 0)),
        out_shape=jax.ShapeDtypeStruct((Nn, Dd, chunks, 128), jnp.float32),
        scratch_shapes=[
            pltpu.VMEM((Nn, Nn, sub, 128), jnp.float32),
            pltpu.VMEM((Nn, Nn, sub, 128), jnp.float32),
        ],
        interpret=interpret,
    )(etv, At, Xt, w0b, w1b, vecb(lin0_b), vecb(lin1_b),
      vecb(ln0_g), vecb(ln0_b), vecb(ln1_g), vecb(ln1_b))

    h = jnp.transpose(out.reshape(Nn, Dd, G), (2, 0, 1))
    return jnp.transpose(h.reshape(Bz, Tt, Nn, Dd), (0, 2, 1, 3))


def kernel(features, adjacency, et_w, et_b, lin0_w, lin0_b, lin1_w, lin1_b,
           ln0_g, ln0_b, ln1_g, ln1_b):
    return _forward(features, adjacency, et_w, et_b, lin0_w, lin0_b,
                    lin1_w, lin1_b, ln0_g, ln0_b, ln1_g, ln1_b)


# speedup vs baseline: 129.4652x; 1.0741x over previous
import jax
import jax.numpy as jnp
from jax.experimental import pallas as pl
from jax.experimental.pallas import tpu as pltpu

_N = 19
_K_EIG = 16
_ALPHA = 0.05
_BETA = (1.0 - _ALPHA) / 2.0
_LAP_EPS = 1e-4
_NSWEEPS = 6
_PAIRS = tuple((p, q) for p in range(_N - 1) for q in range(p + 1, _N))


def _softplus(z):
    return jnp.maximum(z, 0.0) + jnp.log1p(jnp.exp(-jnp.abs(z)))


def _mixer_body(et_ref, A_ref, X_ref, w0_ref, w1_ref, b0_ref, b1_ref,
                g0_ref, be0_ref, g1_ref, be1_ref, out_ref, L_ref, V_ref):
    N = _N
    tile = A_ref.shape[2:]

    A = A_ref[...]
    etw = et_ref[0]
    etb = et_ref[1]

    Aw = jnp.where(A > 0.0, _softplus(A * etw + etb), 0.0)
    degc = jnp.sum(Aw, axis=0)
    disc = jnp.where(degc > 0.0, jax.lax.rsqrt(jnp.maximum(degc, 1e-12)), 0.0)
    An = disc[:, None] * Aw * disc[None, :]

    degr = jnp.sum(A, axis=1)
    disl = jax.lax.rsqrt(jnp.maximum(degr, _LAP_EPS))
    S = disl[:, None] * A * disl[None, :]
    L_ref[...] = -0.5 * (S + jnp.swapaxes(S, 0, 1))
    V_ref[...] = jnp.zeros_like(S)
    for i in range(N):
        L_ref[i, i] = (1.0 + _LAP_EPS) - S[i, i]
        V_ref[i, i] = jnp.ones(tile, jnp.float32)

    def sweep(_, carry):
        for (p, q) in _PAIRS:
            cp = L_ref[:, p]
            cq = L_ref[:, q]
            app = cp[p]
            aqq = cq[q]
            apq = cp[q]
            denom = 2.0 * apq
            safe = jnp.abs(denom) > 1e-37
            tau = (aqq - app) / jnp.where(safe, denom, 1.0)
            tau = jnp.clip(tau, -1e18, 1e18)
            tnum = jnp.where(tau >= 0.0, 1.0, -1.0)
            t = tnum / (jnp.abs(tau) + jnp.sqrt(1.0 + tau * tau))
            t = jnp.where(safe, t, 0.0)
            c = jax.lax.rsqrt(1.0 + t * t)
            s = t * c
            ncp = c * cp - s * cq
            ncq = s * cp + c * cq
            L_ref[:, p] = ncp
            L_ref[:, q] = ncq
            L_ref[p, :] = ncp
            L_ref[q, :] = ncq
            zero = jnp.zeros_like(app)
            L_ref[p, p] = app - t * apq
            L_ref[q, q] = aqq + t * apq
            L_ref[p, q] = zero
            L_ref[q, p] = zero
            vp = V_ref[:, p]
            vq = V_ref[:, q]
            V_ref[:, p] = c * vp - s * vq
            V_ref[:, q] = s * vp + c * vq
        return carry

    jax.lax.fori_loop(0, _NSWEEPS, sweep, 0)

    lam = [L_ref[i, i] for i in range(N)]
    ranks = []
    for i in range(N):
        r = jnp.zeros(tile, jnp.float32)
        for j in range(N):
            if j == i:
                continue
            if j < i:
                r += jnp.where((lam[j] < lam[i]) | (lam[j] == lam[i]), 1.0, 0.0)
            else:
                r += jnp.where(lam[j] < lam[i], 1.0, 0.0)
        ranks.append(r)

    vcols = [V_ref[:, i] for i in range(N)]
    pes = []
    for k in range(_K_EIG):
        acc = jnp.zeros((N,) + tile, jnp.float32)
        for i in range(N):
            acc += jnp.where(ranks[i] == float(k), 1.0, 0.0) * vcols[i]
        ssum = jnp.sum(acc, axis=0)
        sgn = jnp.where(ssum < 0.0, -1.0, 1.0)
        acc = acc * sgn
        acc = jnp.where(jnp.isnan(acc), 0.0, acc)
        pes.append(acc)

    mcols = []
    for r in range(N):
        acc = An[0] * An[r, 0]
        for m in range(1, N):
            acc += An[m] * An[r, m]
        mcols.append(_BETA * (An[r] + acc))

    CH = 8

    def ssg_chunk(zc, bref, jc):
        mv = mcols[0][:, None] * zc[0][None, :]
        for r in range(1, N):
            mv += mcols[r][:, None] * zc[r][None, :]
        return _ALPHA * zc + mv + bref[jc:jc + CH][None]

    def ln(u, gref, bref):
        mu = jnp.mean(u, axis=1, keepdims=True)
        d = u - mu
        var = jnp.mean(d * d, axis=1, keepdims=True)
        return d / jnp.sqrt(var + 1e-5) * gref[...][None] + bref[...][None]

    X = X_ref[...]
    w0 = w0_ref[...]
    D = X.shape[1]
    u0_chunks = []
    for jc in range(0, D, CH):
        zc = X[:, jc:jc + CH]
        for i in range(_K_EIG):
            zc += pes[i][:, None] * w0[i, jc:jc + CH][None]
        u0_chunks.append(ssg_chunk(zc, b0_ref, jc))
    h0 = ln(jnp.concatenate(u0_chunks, axis=1), g0_ref, be0_ref)

    w1 = w1_ref[...]
    u1_chunks = []
    for jc in range(0, D, CH):
        zc = h0[:, 0][:, None] * w1[0, jc:jc + CH][None]
        for i in range(1, D):
            zc += h0[:, i][:, None] * w1[i, jc:jc + CH][None]
        u1_chunks.append(ssg_chunk(zc, b1_ref, jc))
    out_ref[...] = ln(jnp.concatenate(u1_chunks, axis=1),
                      g1_ref, be1_ref) + h0


def _xform_body(x_ref, w_ref, o_ref):
    o_ref[...] = jnp.dot(x_ref[...], w_ref[...],
                         preferred_element_type=jnp.float32)


def _forward(features, adjacency, et_w, et_b, lin0_w, lin0_b, lin1_w, lin1_b,
             ln0_g, ln0_b, ln1_g, ln1_b, interpret=False):
    Bz, Nn, Tt, Dd = features.shape
    G = Bz * Tt
    x = jnp.transpose(features, (0, 2, 1, 3)).reshape(G, Nn, Dd)
    A = adjacency.reshape(G, Nn, Nn)
    chunks = G // 128
    sub = 8
    while chunks % sub:
        sub //= 2
    grid = chunks // sub
    At = jnp.transpose(A, (1, 2, 0)).reshape(Nn, Nn, chunks, 128)

    rows = G * Nn
    rb = 2048
    while rows % rb:
        rb //= 2
    xf = x.reshape(rows, Dd)
    z0x = pl.pallas_call(
        _xform_body,
        grid=(rows // rb,),
        in_specs=[pl.BlockSpec((rb, Dd), lambda i: (i, 0)),
                  pl.BlockSpec((Dd, Dd), lambda i: (0, 0))],
        out_specs=pl.BlockSpec((rb, Dd), lambda i: (i, 0)),
        out_shape=jax.ShapeDtypeStruct((rows, Dd), jnp.float32),
        interpret=interpret,
    )(xf, lin0_w[:Dd])

    Xt = jnp.transpose(z0x.reshape(G, Nn, Dd), (1, 2, 0)).reshape(
        Nn, Dd, chunks, 128)
    w0b = jnp.broadcast_to(lin0_w[Dd:].reshape(_K_EIG, Dd, 1, 1),
                           (_K_EIG, Dd, 1, 128))
    w1b = jnp.broadcast_to(lin1_w.reshape(Dd, Dd, 1, 1), (Dd, Dd, 1, 128))

    def vecb(v):
        return jnp.broadcast_to(v.reshape(Dd, 1, 1), (Dd, 1, 128))

    etv = jnp.concatenate([et_w.reshape(-1), et_b.reshape(-1)]).astype(jnp.float32)

    def cspec(shp):
        nd = len(shp)
        return pl.BlockSpec(shp, lambda i, _n=nd: (0,) * _n)

    out = pl.pallas_call(
        _mixer_body,
        grid=(grid,),
        in_specs=[
            pl.BlockSpec(memory_space=pltpu.SMEM),
            pl.BlockSpec((Nn, Nn, sub, 128), lambda i: (0, 0, i, 0)),
            pl.BlockSpec((Nn, Dd, sub, 128), lambda i: (0, 0, i, 0)),
            cspec((_K_EIG, Dd, 1, 128)),
            cspec((Dd, Dd, 1, 128)),
            cspec((Dd, 1, 128)),
            cspec((Dd, 1, 128)),
            cspec((Dd, 1, 128)),
            cspec((Dd, 1, 128)),
            cspec((Dd, 1, 128)),
            cspec((Dd, 1, 128)),
        ],
        out_specs=pl.BlockSpec((Nn, Dd, sub, 128), lambda i: (0, 0, i, 0)),
        out_shape=jax.ShapeDtypeStruct((Nn, Dd, chunks, 128), jnp.float32),
        scratch_shapes=[
            pltpu.VMEM((Nn, Nn, sub, 128), jnp.float32),
            pltpu.VMEM((Nn, Nn, sub, 128), jnp.float32),
        ],
        interpret=interpret,
    )(etv, At, Xt, w0b, w1b, vecb(lin0_b), vecb(lin1_b),
      vecb(ln0_g), vecb(ln0_b), vecb(ln1_g), vecb(ln1_b))

    h = jnp.transpose(out.reshape(Nn, Dd, G), (2, 0, 1))
    return jnp.transpose(h.reshape(Bz, Tt, Nn, Dd), (0, 2, 1, 3))


def kernel(features, adjacency, et_w, et_b, lin0_w, lin0_b, lin1_w, lin1_b,
           ln0_g, ln0_b, ln1_g, ln1_b):
    return _forward(features, adjacency, et_w, et_b, lin0_w, lin0_b,
                    lin1_w, lin1_b, ln0_g, ln0_b, ln1_g, ln1_b)


# speedup vs baseline: 162.1339x; 1.2523x over previous
import jax
import jax.numpy as jnp
from jax.experimental import pallas as pl
from jax.experimental.pallas import tpu as pltpu

_N = 19
_K_EIG = 16
_ALPHA = 0.05
_BETA = (1.0 - _ALPHA) / 2.0
_LAP_EPS = 1e-4
_NSWEEPS = 6
_PAIRS = tuple((p, q) for p in range(_N - 1) for q in range(p + 1, _N))


def _softplus(z):
    return jnp.maximum(z, 0.0) + jnp.log1p(jnp.exp(-jnp.abs(z)))


def _mixer_body(et_ref, A_ref, X_ref, w0_ref, w1_ref, b0_ref, b1_ref,
                g0_ref, be0_ref, g1_ref, be1_ref, out_ref, L_ref, V_ref):
    N = _N
    tile = A_ref.shape[2:]

    A = A_ref[...]
    etw = et_ref[0]
    etb = et_ref[1]

    Aw = jnp.where(A > 0.0, _softplus(A * etw + etb), 0.0)
    degc = jnp.sum(Aw, axis=0)
    disc = jnp.where(degc > 0.0, jax.lax.rsqrt(jnp.maximum(degc, 1e-12)), 0.0)
    An = disc[:, None] * Aw * disc[None, :]

    degr = jnp.sum(A, axis=1)
    disl = jax.lax.rsqrt(jnp.maximum(degr, _LAP_EPS))
    S = disl[:, None] * A * disl[None, :]
    L_ref[...] = -0.5 * (S + jnp.swapaxes(S, 0, 1))
    V_ref[...] = jnp.zeros_like(S)
    for i in range(N):
        L_ref[i, i] = (1.0 + _LAP_EPS) - S[i, i]
        V_ref[i, i] = jnp.ones(tile, jnp.float32)

    def sweep(_, carry):
        for (p, q) in _PAIRS:
            cp = L_ref[:, p]
            cq = L_ref[:, q]
            app = cp[p]
            aqq = cq[q]
            apq = cp[q]
            denom = 2.0 * apq
            safe = jnp.abs(denom) > 1e-37
            tau = (aqq - app) / jnp.where(safe, denom, 1.0)
            tau = jnp.clip(tau, -1e18, 1e18)
            tnum = jnp.where(tau >= 0.0, 1.0, -1.0)
            t = tnum / (jnp.abs(tau) + jnp.sqrt(1.0 + tau * tau))
            t = jnp.where(safe, t, 0.0)
            c = jax.lax.rsqrt(1.0 + t * t)
            s = t * c
            ncp = c * cp - s * cq
            ncq = s * cp + c * cq
            L_ref[:, p] = ncp
            L_ref[:, q] = ncq
            L_ref[p, :] = ncp
            L_ref[q, :] = ncq
            zero = jnp.zeros_like(app)
            L_ref[p, p] = app - t * apq
            L_ref[q, q] = aqq + t * apq
            L_ref[p, q] = zero
            L_ref[q, p] = zero
            vp = V_ref[:, p]
            vq = V_ref[:, q]
            V_ref[:, p] = c * vp - s * vq
            V_ref[:, q] = s * vp + c * vq
        return carry

    jax.lax.fori_loop(0, _NSWEEPS, sweep, 0)

    lam = [L_ref[i, i] for i in range(N)]
    ranks = []
    for i in range(N):
        r = jnp.zeros(tile, jnp.float32)
        for j in range(N):
            if j == i:
                continue
            if j < i:
                r += jnp.where((lam[j] < lam[i]) | (lam[j] == lam[i]), 1.0, 0.0)
            else:
                r += jnp.where(lam[j] < lam[i], 1.0, 0.0)
        ranks.append(r)

    vcols = [V_ref[:, i] for i in range(N)]
    pes = []
    for k in range(_K_EIG):
        acc = jnp.zeros((N,) + tile, jnp.float32)
        for i in range(N):
            acc += jnp.where(ranks[i] == float(k), 1.0, 0.0) * vcols[i]
        ssum = jnp.sum(acc, axis=0)
        sgn = jnp.where(ssum < 0.0, -1.0, 1.0)
        acc = acc * sgn
        acc = jnp.where(jnp.isnan(acc), 0.0, acc)
        pes.append(acc)

    mcols = []
    for r in range(N):
        acc = An[0] * An[r, 0]
        for m in range(1, N):
            acc += An[m] * An[r, m]
        mcols.append(_BETA * (An[r] + acc))

    CH = 1

    def ssg_chunk(zc, bref, jc):
        mv = mcols[0][:, None] * zc[0][None, :]
        for r in range(1, N):
            mv += mcols[r][:, None] * zc[r][None, :]
        return _ALPHA * zc + mv + bref[jc:jc + CH][None]

    def ln(u, gref, bref):
        mu = jnp.mean(u, axis=1, keepdims=True)
        d = u - mu
        var = jnp.mean(d * d, axis=1, keepdims=True)
        return d / jnp.sqrt(var + 1e-5) * gref[...][None] + bref[...][None]

    X = X_ref[...]
    w0 = w0_ref[...]
    D = X.shape[1]
    u0_chunks = []
    for jc in range(0, D, CH):
        zc = X[:, jc:jc + CH]
        for i in range(_K_EIG):
            zc += pes[i][:, None] * w0[i, jc:jc + CH][None]
        u0_chunks.append(ssg_chunk(zc, b0_ref, jc))
    h0 = ln(jnp.concatenate(u0_chunks, axis=1), g0_ref, be0_ref)

    w1 = w1_ref[...]
    u1_chunks = []
    for jc in range(0, D, CH):
        zc = h0[:, 0][:, None] * w1[0, jc:jc + CH][None]
        for i in range(1, D):
            zc += h0[:, i][:, None] * w1[i, jc:jc + CH][None]
        u1_chunks.append(ssg_chunk(zc, b1_ref, jc))
    out_ref[...] = ln(jnp.concatenate(u1_chunks, axis=1),
                      g1_ref, be1_ref) + h0


def _xform_body(x_ref, w_ref, o_ref):
    o_ref[...] = jnp.dot(x_ref[...], w_ref[...],
                         preferred_element_type=jnp.float32)


def _forward(features, adjacency, et_w, et_b, lin0_w, lin0_b, lin1_w, lin1_b,
             ln0_g, ln0_b, ln1_g, ln1_b, interpret=False):
    Bz, Nn, Tt, Dd = features.shape
    G = Bz * Tt
    x = jnp.transpose(features, (0, 2, 1, 3)).reshape(G, Nn, Dd)
    A = adjacency.reshape(G, Nn, Nn)
    chunks = G // 128
    sub = 8
    while chunks % sub:
        sub //= 2
    grid = chunks // sub
    At = jnp.transpose(A, (1, 2, 0)).reshape(Nn, Nn, chunks, 128)

    rows = G * Nn
    rb = 2048
    while rows % rb:
        rb //= 2
    xf = x.reshape(rows, Dd)
    z0x = pl.pallas_call(
        _xform_body,
        grid=(rows // rb,),
        in_specs=[pl.BlockSpec((rb, Dd), lambda i: (i, 0)),
                  pl.BlockSpec((Dd, Dd), lambda i: (0, 0))],
        out_specs=pl.BlockSpec((rb, Dd), lambda i: (i, 0)),
        out_shape=jax.ShapeDtypeStruct((rows, Dd), jnp.float32),
        interpret=interpret,
    )(xf, lin0_w[:Dd])

    Xt = jnp.transpose(z0x.reshape(G, Nn, Dd), (1, 2, 0)).reshape(
        Nn, Dd, chunks, 128)
    w0b = jnp.broadcast_to(lin0_w[Dd:].reshape(_K_EIG, Dd, 1, 1),
                           (_K_EIG, Dd, 1, 128))
    w1b = jnp.broadcast_to(lin1_w.reshape(Dd, Dd, 1, 1), (Dd, Dd, 1, 128))

    def vecb(v):
        return jnp.broadcast_to(v.reshape(Dd, 1, 1), (Dd, 1, 128))

    etv = jnp.concatenate([et_w.reshape(-1), et_b.reshape(-1)]).astype(jnp.float32)

    def cspec(shp):
        nd = len(shp)
        return pl.BlockSpec(shp, lambda i, _n=nd: (0,) * _n)

    out = pl.pallas_call(
        _mixer_body,
        grid=(grid,),
        in_specs=[
            pl.BlockSpec(memory_space=pltpu.SMEM),
            pl.BlockSpec((Nn, Nn, sub, 128), lambda i: (0, 0, i, 0)),
            pl.BlockSpec((Nn, Dd, sub, 128), lambda i: (0, 0, i, 0)),
            cspec((_K_EIG, Dd, 1, 128)),
            cspec((Dd, Dd, 1, 128)),
            cspec((Dd, 1, 128)),
            cspec((Dd, 1, 128)),
            cspec((Dd, 1, 128)),
            cspec((Dd, 1, 128)),
            cspec((Dd, 1, 128)),
            cspec((Dd, 1, 128)),
        ],
        out_specs=pl.BlockSpec((Nn, Dd, sub, 128), lambda i: (0, 0, i, 0)),
        out_shape=jax.ShapeDtypeStruct((Nn, Dd, chunks, 128), jnp.float32),
        scratch_shapes=[
            pltpu.VMEM((Nn, Nn, sub, 128), jnp.float32),
            pltpu.VMEM((Nn, Nn, sub, 128), jnp.float32),
        ],
        interpret=interpret,
    )(etv, At, Xt, w0b, w1b, vecb(lin0_b), vecb(lin1_b),
      vecb(ln0_g), vecb(ln0_b), vecb(ln1_g), vecb(ln1_b))

    h = jnp.transpose(out.reshape(Nn, Dd, G), (2, 0, 1))
    return jnp.transpose(h.reshape(Bz, Tt, Nn, Dd), (0, 2, 1, 3))


def kernel(features, adjacency, et_w, et_b, lin0_w, lin0_b, lin1_w, lin1_b,
           ln0_g, ln0_b, ln1_g, ln1_b):
    return _forward(features, adjacency, et_w, et_b, lin0_w, lin0_b,
                    lin1_w, lin1_b, ln0_g, ln0_b, ln1_g, ln1_b)


# speedup vs baseline: 221.0666x; 1.3635x over previous
import jax
import jax.numpy as jnp
from jax.experimental import pallas as pl
from jax.experimental.pallas import tpu as pltpu

_N = 19
_K_EIG = 16
_ALPHA = 0.05
_BETA = (1.0 - _ALPHA) / 2.0
_LAP_EPS = 1e-4
_NSWEEPS = 6
_PAIRS = tuple((p, q) for p in range(_N - 1) for q in range(p + 1, _N))


def _softplus(z):
    return jnp.maximum(z, 0.0) + jnp.log1p(jnp.exp(-jnp.abs(z)))


def _mixer_body(et_ref, A_ref, X_ref, w0_ref, w1_ref, b0_ref, b1_ref,
                g0_ref, be0_ref, g1_ref, be1_ref, out_ref, L_ref, V_ref):
    N = _N
    tile = A_ref.shape[2:]

    A = A_ref[...]
    etw = et_ref[0]
    etb = et_ref[1]

    Aw = jnp.where(A > 0.0, _softplus(A * etw + etb), 0.0)
    degc = jnp.sum(Aw, axis=0)
    disc = jnp.where(degc > 0.0, jax.lax.rsqrt(jnp.maximum(degc, 1e-12)), 0.0)
    An = disc[:, None] * Aw * disc[None, :]

    degr = jnp.sum(A, axis=1)
    disl = jax.lax.rsqrt(jnp.maximum(degr, _LAP_EPS))
    S = disl[:, None] * A * disl[None, :]
    L_ref[...] = -0.5 * (S + jnp.swapaxes(S, 0, 1))
    V_ref[...] = jnp.zeros_like(S)
    for i in range(N):
        L_ref[i, i] = (1.0 + _LAP_EPS) - S[i, i]
        V_ref[i, i] = jnp.ones(tile, jnp.float32)

    def sweep(_, carry):
        for (p, q) in _PAIRS:
            cp = L_ref[:, p]
            cq = L_ref[:, q]
            app = cp[p]
            aqq = cq[q]
            apq = cp[q]
            denom = 2.0 * apq
            safe = jnp.abs(denom) > 1e-37
            tau = (aqq - app) / jnp.where(safe, denom, 1.0)
            tau = jnp.clip(tau, -1e18, 1e18)
            tnum = jnp.where(tau >= 0.0, 1.0, -1.0)
            t = tnum / (jnp.abs(tau) + jnp.sqrt(1.0 + tau * tau))
            t = jnp.where(safe, t, 0.0)
            c = jax.lax.rsqrt(1.0 + t * t)
            s = t * c
            ncp = c * cp - s * cq
            ncq = s * cp + c * cq
            L_ref[:, p] = ncp
            L_ref[:, q] = ncq
            L_ref[p, :] = ncp
            L_ref[q, :] = ncq
            zero = jnp.zeros_like(app)
            L_ref[p, p] = app - t * apq
            L_ref[q, q] = aqq + t * apq
            L_ref[p, q] = zero
            L_ref[q, p] = zero
            vp = V_ref[:, p]
            vq = V_ref[:, q]
            V_ref[:, p] = c * vp - s * vq
            V_ref[:, q] = s * vp + c * vq
        return carry

    jax.lax.fori_loop(0, _NSWEEPS, sweep, 0)

    lam = [L_ref[i, i] for i in range(N)]
    ranks = []
    for i in range(N):
        r = jnp.zeros(tile, jnp.float32)
        for j in range(N):
            if j == i:
                continue
            if j < i:
                r += jnp.where((lam[j] < lam[i]) | (lam[j] == lam[i]), 1.0, 0.0)
            else:
                r += jnp.where(lam[j] < lam[i], 1.0, 0.0)
        ranks.append(r)

    vcols = [V_ref[:, i] for i in range(N)]
    pes = []
    for k in range(_K_EIG):
        acc = jnp.zeros((N,) + tile, jnp.float32)
        for i in range(N):
            acc += jnp.where(ranks[i] == float(k), 1.0, 0.0) * vcols[i]
        ssum = jnp.sum(acc, axis=0)
        sgn = jnp.where(ssum < 0.0, -1.0, 1.0)
        acc = acc * sgn
        acc = jnp.where(jnp.isnan(acc), 0.0, acc)
        pes.append(acc)

    mcols = []
    for r in range(N):
        acc = An[0] * An[r, 0]
        for m in range(1, N):
            acc += An[m] * An[r, m]
        mcols.append(_BETA * (An[r] + acc))

    CH = 1

    def ssg_chunk(zc, bref, jc):
        mv = mcols[0][:, None] * zc[0][None, :]
        for r in range(1, N):
            mv += mcols[r][:, None] * zc[r][None, :]
        return _ALPHA * zc + mv + bref[jc:jc + CH][None]

    def ln(u, gref, bref):
        mu = jnp.mean(u, axis=1, keepdims=True)
        d = u - mu
        var = jnp.mean(d * d, axis=1, keepdims=True)
        return d / jnp.sqrt(var + 1e-5) * gref[...][None] + bref[...][None]

    X = X_ref[...]
    w0 = w0_ref[...]
    D = X.shape[1]
    u0_chunks = []
    for jc in range(0, D, CH):
        zc = X[:, jc:jc + CH]
        for i in range(_K_EIG):
            zc += pes[i][:, None] * w0[i, jc:jc + CH][None]
        u0_chunks.append(ssg_chunk(zc, b0_ref, jc))
    h0 = ln(jnp.concatenate(u0_chunks, axis=1), g0_ref, be0_ref)

    w1 = w1_ref[...]
    u1_chunks = []
    for jc in range(0, D, CH):
        zc = h0[:, 0][:, None] * w1[0, jc:jc + CH][None]
        for i in range(1, D):
            zc += h0[:, i][:, None] * w1[i, jc:jc + CH][None]
        u1_chunks.append(ssg_chunk(zc, b1_ref, jc))
    out_ref[...] = ln(jnp.concatenate(u1_chunks, axis=1),
                      g1_ref, be1_ref) + h0


def _xform_body(x_ref, w_ref, o_ref):
    w = w_ref[...]
    for s in range(x_ref.shape[0]):
        for n in range(x_ref.shape[1]):
            o_ref[n, :, s, :] = jax.lax.dot_general(
                w, x_ref[s, n], (((0,), (1,)), ((), ())),
                preferred_element_type=jnp.float32)


def _forward(features, adjacency, et_w, et_b, lin0_w, lin0_b, lin1_w, lin1_b,
             ln0_g, ln0_b, ln1_g, ln1_b, interpret=False):
    Bz, Nn, Tt, Dd = features.shape
    chunks = Bz
    sub = 8
    while chunks % sub:
        sub //= 2
    grid = chunks // sub
    At = jnp.transpose(adjacency, (2, 3, 0, 1))

    Xt = pl.pallas_call(
        _xform_body,
        grid=(Bz // sub,),
        in_specs=[pl.BlockSpec((sub, Nn, Tt, Dd), lambda b: (b, 0, 0, 0)),
                  pl.BlockSpec((Dd, Dd), lambda b: (0, 0))],
        out_specs=pl.BlockSpec((Nn, Dd, sub, Tt), lambda b: (0, 0, b, 0)),
        out_shape=jax.ShapeDtypeStruct((Nn, Dd, Bz, Tt), jnp.float32),
        interpret=interpret,
    )(features, lin0_w[:Dd])
    w0b = jnp.broadcast_to(lin0_w[Dd:].reshape(_K_EIG, Dd, 1, 1),
                           (_K_EIG, Dd, 1, 128))
    w1b = jnp.broadcast_to(lin1_w.reshape(Dd, Dd, 1, 1), (Dd, Dd, 1, 128))

    def vecb(v):
        return jnp.broadcast_to(v.reshape(Dd, 1, 1), (Dd, 1, 128))

    etv = jnp.concatenate([et_w.reshape(-1), et_b.reshape(-1)]).astype(jnp.float32)

    def cspec(shp):
        nd = len(shp)
        return pl.BlockSpec(shp, lambda i, _n=nd: (0,) * _n)

    out = pl.pallas_call(
        _mixer_body,
        grid=(grid,),
        in_specs=[
            pl.BlockSpec(memory_space=pltpu.SMEM),
            pl.BlockSpec((Nn, Nn, sub, 128), lambda i: (0, 0, i, 0)),
            pl.BlockSpec((Nn, Dd, sub, 128), lambda i: (0, 0, i, 0)),
            cspec((_K_EIG, Dd, 1, 128)),
            cspec((Dd, Dd, 1, 128)),
            cspec((Dd, 1, 128)),
            cspec((Dd, 1, 128)),
            cspec((Dd, 1, 128)),
            cspec((Dd, 1, 128)),
            cspec((Dd, 1, 128)),
            cspec((Dd, 1, 128)),
        ],
        out_specs=pl.BlockSpec((Nn, Dd, sub, 128), lambda i: (0, 0, i, 0)),
        out_shape=jax.ShapeDtypeStruct((Nn, Dd, chunks, 128), jnp.float32),
        scratch_shapes=[
            pltpu.VMEM((Nn, Nn, sub, 128), jnp.float32),
            pltpu.VMEM((Nn, Nn, sub, 128), jnp.float32),
        ],
        interpret=interpret,
    )(etv, At, Xt, w0b, w1b, vecb(lin0_b), vecb(lin1_b),
      vecb(ln0_g), vecb(ln0_b), vecb(ln1_g), vecb(ln1_b))

    return jnp.transpose(out, (2, 0, 3, 1))


def kernel(features, adjacency, et_w, et_b, lin0_w, lin0_b, lin1_w, lin1_b,
           ln0_g, ln0_b, ln1_g, ln1_b):
    return _forward(features, adjacency, et_w, et_b, lin0_w, lin0_b,
                    lin1_w, lin1_b, ln0_g, ln0_b, ln1_g, ln1_b)


# speedup vs baseline: 233.9079x; 1.0581x over previous
import jax
import jax.numpy as jnp
from jax.experimental import pallas as pl
from jax.experimental.pallas import tpu as pltpu

_N = 19
_K_EIG = 16
_ALPHA = 0.05
_BETA = (1.0 - _ALPHA) / 2.0
_LAP_EPS = 1e-4
_NSWEEPS = 5
_PAIRS = tuple((p, q) for p in range(_N - 1) for q in range(p + 1, _N))


def _softplus(z):
    return jnp.maximum(z, 0.0) + jnp.log1p(jnp.exp(-jnp.abs(z)))


def _mixer_body(et_ref, A_ref, X_ref, w0_ref, w1_ref, b0_ref, b1_ref,
                g0_ref, be0_ref, g1_ref, be1_ref, out_ref, L_ref, V_ref):
    N = _N
    tile = A_ref.shape[2:]

    A = A_ref[...]
    etw = et_ref[0]
    etb = et_ref[1]

    Aw = jnp.where(A > 0.0, _softplus(A * etw + etb), 0.0)
    degc = jnp.sum(Aw, axis=0)
    disc = jnp.where(degc > 0.0, jax.lax.rsqrt(jnp.maximum(degc, 1e-12)), 0.0)
    An = disc[:, None] * Aw * disc[None, :]

    degr = jnp.sum(A, axis=1)
    disl = jax.lax.rsqrt(jnp.maximum(degr, _LAP_EPS))
    S = disl[:, None] * A * disl[None, :]
    L_ref[...] = -0.5 * (S + jnp.swapaxes(S, 0, 1))
    V_ref[...] = jnp.zeros_like(S)
    for i in range(N):
        L_ref[i, i] = (1.0 + _LAP_EPS) - S[i, i]
        V_ref[i, i] = jnp.ones(tile, jnp.float32)

    def sweep(_, carry):
        for (p, q) in _PAIRS:
            cp = L_ref[:, p]
            cq = L_ref[:, q]
            app = cp[p]
            aqq = cq[q]
            apq = cp[q]
            denom = 2.0 * apq
            safe = jnp.abs(denom) > 1e-37
            tau = (aqq - app) / jnp.where(safe, denom, 1.0)
            tau = jnp.clip(tau, -1e18, 1e18)
            tnum = jnp.where(tau >= 0.0, 1.0, -1.0)
            t = tnum / (jnp.abs(tau) + jnp.sqrt(1.0 + tau * tau))
            t = jnp.where(safe, t, 0.0)
            c = jax.lax.rsqrt(1.0 + t * t)
            s = t * c
            ncp = c * cp - s * cq
            ncq = s * cp + c * cq
            L_ref[:, p] = ncp
            L_ref[:, q] = ncq
            L_ref[p, :] = ncp
            L_ref[q, :] = ncq
            zero = jnp.zeros_like(app)
            L_ref[p, p] = app - t * apq
            L_ref[q, q] = aqq + t * apq
            L_ref[p, q] = zero
            L_ref[q, p] = zero
            vp = V_ref[:, p]
            vq = V_ref[:, q]
            V_ref[:, p] = c * vp - s * vq
            V_ref[:, q] = s * vp + c * vq
        return carry

    jax.lax.fori_loop(0, _NSWEEPS, sweep, 0)

    lam = [L_ref[i, i] for i in range(N)]
    ranks = []
    for i in range(N):
        r = jnp.zeros(tile, jnp.float32)
        for j in range(N):
            if j == i:
                continue
            if j < i:
                r += jnp.where((lam[j] < lam[i]) | (lam[j] == lam[i]), 1.0, 0.0)
            else:
                r += jnp.where(lam[j] < lam[i], 1.0, 0.0)
        ranks.append(r)

    vcols = [V_ref[:, i] for i in range(N)]
    pes = []
    for k in range(_K_EIG):
        acc = jnp.zeros((N,) + tile, jnp.float32)
        for i in range(N):
            acc += jnp.where(ranks[i] == float(k), 1.0, 0.0) * vcols[i]
        ssum = jnp.sum(acc, axis=0)
        sgn = jnp.where(ssum < 0.0, -1.0, 1.0)
        acc = acc * sgn
        acc = jnp.where(jnp.isnan(acc), 0.0, acc)
        pes.append(acc)

    mcols = []
    for r in range(N):
        acc = An[0] * An[r, 0]
        for m in range(1, N):
            acc += An[m] * An[r, m]
        mcols.append(_BETA * (An[r] + acc))

    CH = 1

    def ssg_chunk(zc, bref, jc):
        mv = mcols[0][:, None] * zc[0][None, :]
        for r in range(1, N):
            mv += mcols[r][:, None] * zc[r][None, :]
        return _ALPHA * zc + mv + bref[jc:jc + CH][None]

    def ln(u, gref, bref):
        mu = jnp.mean(u, axis=1, keepdims=True)
        d = u - mu
        var = jnp.mean(d * d, axis=1, keepdims=True)
        return d / jnp.sqrt(var + 1e-5) * gref[...][None] + bref[...][None]

    X = X_ref[...]
    w0 = w0_ref[...]
    D = X.shape[1]
    u0_chunks = []
    for jc in range(0, D, CH):
        zc = X[:, jc:jc + CH]
        for i in range(_K_EIG):
            zc += pes[i][:, None] * w0[i, jc:jc + CH][None]
        u0_chunks.append(ssg_chunk(zc, b0_ref, jc))
    h0 = ln(jnp.concatenate(u0_chunks, axis=1), g0_ref, be0_ref)

    w1 = w1_ref[...]
    u1_chunks = []
    for jc in range(0, D, CH):
        zc = h0[:, 0][:, None] * w1[0, jc:jc + CH][None]
        for i in range(1, D):
            zc += h0[:, i][:, None] * w1[i, jc:jc + CH][None]
        u1_chunks.append(ssg_chunk(zc, b1_ref, jc))
    out_ref[...] = ln(jnp.concatenate(u1_chunks, axis=1),
                      g1_ref, be1_ref) + h0


def _xform_body(x_ref, w_ref, o_ref):
    w = w_ref[...]
    for s in range(x_ref.shape[0]):
        for n in range(x_ref.shape[1]):
            o_ref[n, :, s, :] = jax.lax.dot_general(
                w, x_ref[s, n], (((0,), (1,)), ((), ())),
                preferred_element_type=jnp.float32)


def _forward(features, adjacency, et_w, et_b, lin0_w, lin0_b, lin1_w, lin1_b,
             ln0_g, ln0_b, ln1_g, ln1_b, interpret=False):
    Bz, Nn, Tt, Dd = features.shape
    chunks = Bz
    sub = 8
    while chunks % sub:
        sub //= 2
    grid = chunks // sub
    At = jnp.transpose(adjacency, (2, 3, 0, 1))

    Xt = pl.pallas_call(
        _xform_body,
        grid=(Bz // sub,),
        in_specs=[pl.BlockSpec((sub, Nn, Tt, Dd), lambda b: (b, 0, 0, 0)),
                  pl.BlockSpec((Dd, Dd), lambda b: (0, 0))],
        out_specs=pl.BlockSpec((Nn, Dd, sub, Tt), lambda b: (0, 0, b, 0)),
        out_shape=jax.ShapeDtypeStruct((Nn, Dd, Bz, Tt), jnp.float32),
        interpret=interpret,
    )(features, lin0_w[:Dd])
    w0b = jnp.broadcast_to(lin0_w[Dd:].reshape(_K_EIG, Dd, 1, 1),
                           (_K_EIG, Dd, 1, 128))
    w1b = jnp.broadcast_to(lin1_w.reshape(Dd, Dd, 1, 1), (Dd, Dd, 1, 128))

    def vecb(v):
        return jnp.broadcast_to(v.reshape(Dd, 1, 1), (Dd, 1, 128))

    etv = jnp.concatenate([et_w.reshape(-1), et_b.reshape(-1)]).astype(jnp.float32)

    def cspec(shp):
        nd = len(shp)
        return pl.BlockSpec(shp, lambda i, _n=nd: (0,) * _n)

    out = pl.pallas_call(
        _mixer_body,
        grid=(grid,),
        in_specs=[
            pl.BlockSpec(memory_space=pltpu.SMEM),
            pl.BlockSpec((Nn, Nn, sub, 128), lambda i: (0, 0, i, 0)),
            pl.BlockSpec((Nn, Dd, sub, 128), lambda i: (0, 0, i, 0)),
            cspec((_K_EIG, Dd, 1, 128)),
            cspec((Dd, Dd, 1, 128)),
            cspec((Dd, 1, 128)),
            cspec((Dd, 1, 128)),
            cspec((Dd, 1, 128)),
            cspec((Dd, 1, 128)),
            cspec((Dd, 1, 128)),
            cspec((Dd, 1, 128)),
        ],
        out_specs=pl.BlockSpec((Nn, Dd, sub, 128), lambda i: (0, 0, i, 0)),
        out_shape=jax.ShapeDtypeStruct((Nn, Dd, chunks, 128), jnp.float32),
        scratch_shapes=[
            pltpu.VMEM((Nn, Nn, sub, 128), jnp.float32),
            pltpu.VMEM((Nn, Nn, sub, 128), jnp.float32),
        ],
        interpret=interpret,
    )(etv, At, Xt, w0b, w1b, vecb(lin0_b), vecb(lin1_b),
      vecb(ln0_g), vecb(ln0_b), vecb(ln1_g), vecb(ln1_b))

    return jnp.transpose(out, (2, 0, 3, 1))


def kernel(features, adjacency, et_w, et_b, lin0_w, lin0_b, lin1_w, lin1_b,
           ln0_g, ln0_b, ln1_g, ln1_b):
    return _forward(features, adjacency, et_w, et_b, lin0_w, lin0_b,
                    lin1_w, lin1_b, ln0_g, ln0_b, ln1_g, ln1_b)
